# Initial kernel scaffold; baseline (speedup 1.0000x reference)
import functools

import jax
import jax.numpy as jnp
from jax import lax
from jax.experimental import pallas as pl
from jax.experimental.pallas import tpu as pltpu
from jax.experimental.pallas import tpu_sc as plsc

N = 100000
E = 3200000
F = 16
FA = 8
FO = 32

NC = 2
NS = 16
NW = NC * NS
E_PER_W = E // NW
CHUNK = 2000
N_CHUNKS = E_PER_W // CHUNK
ROWS_PER_TILE = N // NS

BE = 20000
BN = 10000


def _edge_mlp_body(a_ref, w_ref, b_ref, o_ref):
    o_ref[...] = (
        lax.dot_general(a_ref[...], w_ref[...], (((1,), (0,)), ((), ())),
                        preferred_element_type=jnp.float32)
        + b_ref[...]
    )


def _edge_mlp(edge_attr, w_t, b):
    return pl.pallas_call(
        _edge_mlp_body,
        grid=(E // BE,),
        in_specs=[
            pl.BlockSpec((BE, FA), lambda i: (i, 0)),
            pl.BlockSpec((FA, F), lambda i: (0, 0)),
            pl.BlockSpec((1, F), lambda i: (0, 0)),
        ],
        out_specs=pl.BlockSpec((BE, F), lambda i: (i, 0)),
        out_shape=jax.ShapeDtypeStruct((E, F), jnp.float32),
    )(edge_attr, w_t, b)


def _final_body(x_ref, a_ref, w_ref, b_ref, o_ref):
    h = x_ref[...] + a_ref[0] + a_ref[1]
    o_ref[...] = (
        lax.dot_general(h, w_ref[...], (((1,), (0,)), ((), ())),
                        preferred_element_type=jnp.float32)
        + b_ref[...]
    )


def _final(x, aggr2, w_t, b):
    return pl.pallas_call(
        _final_body,
        grid=(N // BN,),
        in_specs=[
            pl.BlockSpec((BN, F), lambda i: (i, 0)),
            pl.BlockSpec((NC, BN, F), lambda i: (0, i, 0)),
            pl.BlockSpec((F, FO), lambda i: (0, 0)),
            pl.BlockSpec((1, FO), lambda i: (0, 0)),
        ],
        out_specs=pl.BlockSpec((BN, FO), lambda i: (i, 0)),
        out_shape=jax.ShapeDtypeStruct((N, FO), jnp.float32),
    )(x, aggr2, w_t, b)


def _sc_body(src_hbm, dst_hbm, e_hbm, x_hbm, out_hbm,
             src_v, dst_v, e_v, xg_v, aggr_sh, sem):
    c = lax.axis_index("c")
    s = lax.axis_index("s")
    wid = c * NS + s

    def zero_body(i, _):
        e_v[i] = jnp.zeros((F,), jnp.float32)
        return 0

    lax.fori_loop(0, CHUNK, zero_body, 0)
    nbase = s * ROWS_PER_TILE
    for j in range(ROWS_PER_TILE // CHUNK):
        pltpu.sync_copy(e_v, aggr_sh.at[pl.ds(nbase + j * CHUNK, CHUNK)])
    rem = ROWS_PER_TILE % CHUNK
    if rem:
        pltpu.sync_copy(
            e_v.at[pl.ds(0, rem)],
            aggr_sh.at[pl.ds(nbase + (ROWS_PER_TILE // CHUNK) * CHUNK, rem)])
    plsc.subcore_barrier()

    ebase = wid * E_PER_W

    def chunk_body(g, _):
        off = ebase + g * CHUNK
        pltpu.sync_copy(src_hbm.at[pl.ds(off, CHUNK)], src_v)
        pltpu.sync_copy(dst_hbm.at[pl.ds(off, CHUNK)], dst_v)
        pltpu.sync_copy(e_hbm.at[pl.ds(off, CHUNK)], e_v)
        pltpu.async_copy(x_hbm.at[src_v], xg_v, sem).wait()

        def edge_body(i, _):
            xg_v[i] = jnp.maximum(xg_v[i] + e_v[i], 0.0)
            return 0

        lax.fori_loop(0, CHUNK, edge_body, 0)
        pltpu.sync_copy(xg_v, aggr_sh.at[dst_v], add=True)
        return 0

    lax.fori_loop(0, N_CHUNKS, chunk_body, 0)
    plsc.subcore_barrier()

    pltpu.sync_copy(aggr_sh.at[pl.ds(nbase, ROWS_PER_TILE)],
                    out_hbm.at[c].at[pl.ds(nbase, ROWS_PER_TILE)])


_sc_aggregate = functools.partial(
    pl.kernel,
    out_type=jax.ShapeDtypeStruct((NC, N, F), jnp.float32),
    mesh=plsc.VectorSubcoreMesh(core_axis_name="c", subcore_axis_name="s"),
    scratch_types=[
        pltpu.VMEM((CHUNK,), jnp.int32),
        pltpu.VMEM((CHUNK,), jnp.int32),
        pltpu.VMEM((CHUNK, F), jnp.float32),
        pltpu.VMEM((CHUNK, F), jnp.float32),
        pltpu.VMEM_SHARED((N, F), jnp.float32),
        pltpu.SemaphoreType.DMA,
    ],
)(_sc_body)


def kernel(x, edge_index, edge_attr, lin1_W, lin1_b, nn_W, nn_b):
    src = edge_index[0].astype(jnp.int32)
    dst = edge_index[1].astype(jnp.int32)
    e = _edge_mlp(edge_attr, lin1_W.T, lin1_b.reshape(1, F))
    aggr2 = _sc_aggregate(src, dst, e, x)
    out = _final(x, aggr2, nn_W.T, nn_b.reshape(1, FO))
    return out


# speedup vs baseline: 5.5749x; 5.5749x over previous
import functools

import jax
import jax.numpy as jnp
from jax import lax
from jax.experimental import pallas as pl
from jax.experimental.pallas import tpu as pltpu
from jax.experimental.pallas import tpu_sc as plsc

N = 100000
E = 3200000
F = 16
FA = 8
FO = 32

NC = 2
NS = 16
NW = NC * NS
E_PER_W = E // NW
CHUNK = 800
N_CHUNKS = E_PER_W // CHUNK
ROWS_PER_TILE = 6256
N_PAD = ROWS_PER_TILE * NS

BE = 20000
BN = 10000


def _edge_mlp_body(a_ref, w_ref, b_ref, o_ref):
    o_ref[...] = (
        lax.dot_general(a_ref[...], w_ref[...], (((1,), (0,)), ((), ())),
                        preferred_element_type=jnp.float32)
        + b_ref[...]
    )


def _edge_mlp(edge_attr, w_t, b):
    return pl.pallas_call(
        _edge_mlp_body,
        grid=(E // BE,),
        in_specs=[
            pl.BlockSpec((BE, FA), lambda i: (i, 0)),
            pl.BlockSpec((FA, F), lambda i: (0, 0)),
            pl.BlockSpec((1, F), lambda i: (0, 0)),
        ],
        out_specs=pl.BlockSpec((BE, F), lambda i: (i, 0)),
        out_shape=jax.ShapeDtypeStruct((E, F), jnp.float32),
    )(edge_attr, w_t, b)


def _final_body(x_ref, a_ref, w_ref, b_ref, o_ref):
    h = x_ref[...] + a_ref[0] + a_ref[1]
    o_ref[...] = (
        lax.dot_general(h, w_ref[...], (((1,), (0,)), ((), ())),
                        preferred_element_type=jnp.float32)
        + b_ref[...]
    )


def _final(x, aggr2, w_t, b):
    return pl.pallas_call(
        _final_body,
        grid=(N // BN,),
        in_specs=[
            pl.BlockSpec((BN, F), lambda i: (i, 0)),
            pl.BlockSpec((NC, BN, F), lambda i: (0, i, 0)),
            pl.BlockSpec((F, FO), lambda i: (0, 0)),
            pl.BlockSpec((1, FO), lambda i: (0, 0)),
        ],
        out_specs=pl.BlockSpec((BN, FO), lambda i: (i, 0)),
        out_shape=jax.ShapeDtypeStruct((N, FO), jnp.float32),
    )(x, aggr2, w_t, b)


def _sc_body(src_hbm, dst_hbm, e_hbm, x_hbm, out_hbm,
             src_v, dst_v, e_v, xg_v, aggr_sh, sem):
    c = lax.axis_index("c")
    s = lax.axis_index("s")
    wid = c * NS + s

    def zero_body(i, _):
        e_v[i] = jnp.zeros((F,), jnp.float32)
        return 0

    lax.fori_loop(0, CHUNK, zero_body, 0)
    nbase = s * ROWS_PER_TILE
    for j in range(ROWS_PER_TILE // CHUNK):
        pltpu.sync_copy(e_v, aggr_sh.at[pl.ds(nbase + j * CHUNK, CHUNK)])
    rem = ROWS_PER_TILE % CHUNK
    if rem:
        pltpu.sync_copy(
            e_v.at[pl.ds(0, rem)],
            aggr_sh.at[pl.ds(nbase + (ROWS_PER_TILE // CHUNK) * CHUNK, rem)])
    plsc.subcore_barrier()

    ebase = wid * E_PER_W

    def chunk_body(g, _):
        off = ebase + g * CHUNK
        pltpu.sync_copy(src_hbm.at[pl.ds(off, CHUNK)], src_v)
        pltpu.sync_copy(dst_hbm.at[pl.ds(off, CHUNK)], dst_v)
        pltpu.sync_copy(e_hbm.at[pl.ds(off, CHUNK)], e_v)
        pltpu.async_copy(x_hbm.at[src_v], xg_v, sem).wait()

        def edge_body(i, _):
            xg_v[i] = jnp.maximum(xg_v[i] + e_v[i], 0.0)
            return 0

        lax.fori_loop(0, CHUNK, edge_body, 0)
        pltpu.sync_copy(xg_v, aggr_sh.at[dst_v], add=True)
        return 0

    lax.fori_loop(0, N_CHUNKS, chunk_body, 0)
    plsc.subcore_barrier()

    pltpu.sync_copy(aggr_sh.at[pl.ds(nbase, ROWS_PER_TILE)],
                    out_hbm.at[c].at[pl.ds(nbase, ROWS_PER_TILE)])


_sc_aggregate = functools.partial(
    pl.kernel,
    out_type=jax.ShapeDtypeStruct((NC, N_PAD, F), jnp.float32),
    mesh=plsc.VectorSubcoreMesh(core_axis_name="c", subcore_axis_name="s",
                                num_cores=NC, num_subcores=NS),
    compiler_params=pltpu.CompilerParams(use_tc_tiling_on_sc=False),
    scratch_types=[
        pltpu.VMEM((CHUNK,), jnp.int32),
        pltpu.VMEM((CHUNK,), jnp.int32),
        pltpu.VMEM((CHUNK, F), jnp.float32),
        pltpu.VMEM((CHUNK, F), jnp.float32),
        pltpu.VMEM_SHARED((N_PAD, F), jnp.float32),
        pltpu.SemaphoreType.DMA,
    ],
)(_sc_body)


def kernel(x, edge_index, edge_attr, lin1_W, lin1_b, nn_W, nn_b):
    src = edge_index[0].astype(jnp.int32)
    dst = edge_index[1].astype(jnp.int32)
    e = _edge_mlp(edge_attr, lin1_W.T, lin1_b.reshape(1, F))
    aggr2 = _sc_aggregate(src, dst, e, x)[:, :N, :]
    out = _final(x, aggr2, nn_W.T, nn_b.reshape(1, FO))
    return out


# speedup vs baseline: 8.7489x; 1.5693x over previous
import functools

import jax
import jax.numpy as jnp
from jax import lax
from jax.experimental import pallas as pl
from jax.experimental.pallas import tpu as pltpu
from jax.experimental.pallas import tpu_sc as plsc

N = 100000
E = 3200000
F = 16
FA = 8
FO = 32

NC = 2
NS = 16
NW = NC * NS
E_PER_W = E // NW
CHUNK = 400
PAIRS = CHUNK // 2
N_CHUNKS = E_PER_W // CHUNK
ROWS_PER_TILE = 6256
N_PAD = ROWS_PER_TILE * NS

BN = 10000


def _final_body(x_ref, a_ref, w_ref, b_ref, o_ref):
    h = x_ref[...] + a_ref[0] + a_ref[1]
    o_ref[...] = (
        lax.dot_general(h, w_ref[...], (((1,), (0,)), ((), ())),
                        preferred_element_type=jnp.float32)
        + b_ref[...]
    )


def _final(x, aggr2, w_t, b):
    return pl.pallas_call(
        _final_body,
        grid=(N // BN,),
        in_specs=[
            pl.BlockSpec((BN, F), lambda i: (i, 0)),
            pl.BlockSpec((NC, BN, F), lambda i: (0, i, 0)),
            pl.BlockSpec((F, FO), lambda i: (0, 0)),
            pl.BlockSpec((1, FO), lambda i: (0, 0)),
        ],
        out_specs=pl.BlockSpec((BN, FO), lambda i: (i, 0)),
        out_shape=jax.ShapeDtypeStruct((N, FO), jnp.float32),
    )(x, aggr2, w_t, b)


def _in_copies(eidx_hbm, attr_hbm, src_v, dst_v, attr_v, sem, off):
    return (
        pltpu.make_async_copy(
            eidx_hbm.at[0].at[pl.ds(off, CHUNK)], src_v, sem),
        pltpu.make_async_copy(
            eidx_hbm.at[1].at[pl.ds(off, CHUNK)], dst_v, sem),
        pltpu.make_async_copy(
            attr_hbm.at[pl.ds(off // 2, PAIRS)], attr_v, sem),
    )


def _sc_body(eidx_hbm, attr_hbm, x_hbm, w_hbm, b_hbm, out_hbm,
             src0_v, src1_v, dst0_v, dst1_v, attr_v, xg_v, wb_v, aggr_sh,
             sem_in0, sem_in1, sem_g):
    c = lax.axis_index("c")
    s = lax.axis_index("s")
    wid = c * NS + s
    sem_in = (sem_in0, sem_in1)
    src_b = (src0_v, src1_v)
    dst_b = (dst0_v, dst1_v)

    pltpu.sync_copy(w_hbm, wb_v.at[pl.ds(0, FA)])
    pltpu.sync_copy(b_hbm, wb_v.at[FA])
    wr = [wb_v[k] for k in range(FA)]
    bvec = wb_v[FA]

    def zero_body(i, _):
        xg_v[0, i] = jnp.zeros((F,), jnp.float32)
        return 0

    lax.fori_loop(0, CHUNK, zero_body, 0)
    nbase = s * ROWS_PER_TILE
    for j in range(ROWS_PER_TILE // CHUNK):
        pltpu.sync_copy(xg_v.at[0], aggr_sh.at[pl.ds(nbase + j * CHUNK, CHUNK)])
    rem = ROWS_PER_TILE % CHUNK
    if rem:
        pltpu.sync_copy(
            xg_v.at[0].at[pl.ds(0, rem)],
            aggr_sh.at[pl.ds(nbase + (ROWS_PER_TILE // CHUNK) * CHUNK, rem)])
    plsc.subcore_barrier()

    ebase = wid * E_PER_W

    ksel = [jnp.full((F,), k, jnp.int32) for k in range(2 * FA)]

    for d in _in_copies(eidx_hbm, attr_hbm, src_b[0], dst_b[0],
                        attr_v.at[0], sem_in[0], ebase):
        d.start()

    def compute_chunk(b):
        def pair_body(p, _):
            a2 = attr_v[b, p]
            i0 = 2 * p
            i1 = 2 * p + 1
            acc0 = xg_v[b, i0] + bvec
            acc1 = xg_v[b, i1] + bvec
            for k in range(FA):
                acc0 = acc0 + a2[ksel[k]] * wr[k]
                acc1 = acc1 + a2[ksel[FA + k]] * wr[k]
            xg_v[b, i0] = jnp.maximum(acc0, 0.0)
            xg_v[b, i1] = jnp.maximum(acc1, 0.0)
            return 0

        lax.fori_loop(0, PAIRS, pair_body, 0)

    def outer_body(g2, _):
        for b in range(2):
            g = 2 * g2 + b
            off = ebase + g * CHUNK
            bo = 1 - b
            for d in _in_copies(eidx_hbm, attr_hbm, src_b[b], dst_b[b],
                                attr_v.at[b], sem_in[b], off):
                d.wait()
            @pl.when(g + 1 < N_CHUNKS)
            def _():
                for d in _in_copies(eidx_hbm, attr_hbm, src_b[bo], dst_b[bo],
                                    attr_v.at[bo], sem_in[bo], off + CHUNK):
                    d.start()
            pltpu.async_copy(x_hbm.at[src_b[b]], xg_v.at[b], sem_g).wait()
            compute_chunk(b)
            pltpu.sync_copy(xg_v.at[b], aggr_sh.at[dst_b[b]], add=True)
        return 0

    lax.fori_loop(0, N_CHUNKS // 2, outer_body, 0)
    plsc.subcore_barrier()

    pltpu.sync_copy(aggr_sh.at[pl.ds(nbase, ROWS_PER_TILE)],
                    out_hbm.at[c].at[pl.ds(nbase, ROWS_PER_TILE)])


_sc_aggregate = functools.partial(
    pl.kernel,
    out_type=jax.ShapeDtypeStruct((NC, N_PAD, F), jnp.float32),
    mesh=plsc.VectorSubcoreMesh(core_axis_name="c", subcore_axis_name="s",
                                num_cores=NC, num_subcores=NS),
    compiler_params=pltpu.CompilerParams(use_tc_tiling_on_sc=False),
    scratch_types=[
        pltpu.VMEM((CHUNK,), jnp.int32),
        pltpu.VMEM((CHUNK,), jnp.int32),
        pltpu.VMEM((CHUNK,), jnp.int32),
        pltpu.VMEM((CHUNK,), jnp.int32),
        pltpu.VMEM((2, PAIRS, 2 * FA), jnp.float32),
        pltpu.VMEM((2, CHUNK, F), jnp.float32),
        pltpu.VMEM((FA + 1, F), jnp.float32),
        pltpu.VMEM_SHARED((N_PAD, F), jnp.float32),
        pltpu.SemaphoreType.DMA,
        pltpu.SemaphoreType.DMA,
        pltpu.SemaphoreType.DMA,
    ],
)(_sc_body)


def kernel(x, edge_index, edge_attr, lin1_W, lin1_b, nn_W, nn_b):
    eidx = edge_index.astype(jnp.int32)
    attr2 = edge_attr.reshape(E // 2, 2 * FA)
    aggr2 = _sc_aggregate(eidx, attr2, x, lin1_W.T, lin1_b)[:, :N, :]
    out = _final(x, aggr2, nn_W.T, nn_b.reshape(1, FO))
    return out


# speedup vs baseline: 9.0059x; 1.0294x over previous
import functools

import jax
import jax.numpy as jnp
from jax import lax
from jax.experimental import pallas as pl
from jax.experimental.pallas import tpu as pltpu
from jax.experimental.pallas import tpu_sc as plsc

N = 100000
E = 3200000
F = 16
FA = 8
FO = 32

NC = 2
NS = 16
NW = NC * NS
E_PER_W = E // NW
CHUNK = 400
PAIRS = CHUNK // 2
N_CHUNKS = E_PER_W // CHUNK
ROWS_PER_TILE = 6256
N_PAD = ROWS_PER_TILE * NS

BN = 10000


def _final_body(x_ref, a_ref, w_ref, b_ref, o_ref):
    h = x_ref[...] + a_ref[0] + a_ref[1]
    o_ref[...] = (
        lax.dot_general(h, w_ref[...], (((1,), (0,)), ((), ())),
                        preferred_element_type=jnp.float32)
        + b_ref[...]
    )


def _final(x, aggr2, w_t, b):
    return pl.pallas_call(
        _final_body,
        grid=(N // BN,),
        in_specs=[
            pl.BlockSpec((BN, F), lambda i: (i, 0)),
            pl.BlockSpec((NC, BN, F), lambda i: (0, i, 0)),
            pl.BlockSpec((F, FO), lambda i: (0, 0)),
            pl.BlockSpec((1, FO), lambda i: (0, 0)),
        ],
        out_specs=pl.BlockSpec((BN, FO), lambda i: (i, 0)),
        out_shape=jax.ShapeDtypeStruct((N, FO), jnp.float32),
    )(x, aggr2, w_t, b)


def _in_copies(eidx_hbm, attr_hbm, src_v, dst_v, attr_v, sem, off):
    return (
        pltpu.make_async_copy(
            eidx_hbm.at[0].at[pl.ds(off, CHUNK)], src_v, sem),
        pltpu.make_async_copy(
            eidx_hbm.at[1].at[pl.ds(off, CHUNK)], dst_v, sem),
        pltpu.make_async_copy(
            attr_hbm.at[pl.ds(off * FA, CHUNK * FA)], attr_v, sem),
    )


def _sc_body(eidx_hbm, attr_hbm, x_hbm, w_hbm, b_hbm, out_hbm,
             src0_v, src1_v, dst0_v, dst1_v, attr0_v, attr1_v, xg_v, wb_v,
             aggr_sh, sem_in0, sem_in1, sem_g0, sem_g1):
    c = lax.axis_index("c")
    s = lax.axis_index("s")
    wid = c * NS + s
    sem_in = (sem_in0, sem_in1)
    sem_g = (sem_g0, sem_g1)
    src_b = (src0_v, src1_v)
    dst_b = (dst0_v, dst1_v)
    attr_b = (attr0_v, attr1_v)

    pltpu.sync_copy(w_hbm, wb_v.at[pl.ds(0, FA)])
    pltpu.sync_copy(b_hbm, wb_v.at[FA])
    wr = [wb_v[k] for k in range(FA)]
    bvec = wb_v[FA]

    def zero_body(i, _):
        xg_v[0, i] = jnp.zeros((F,), jnp.float32)
        return 0

    lax.fori_loop(0, CHUNK, zero_body, 0)
    nbase = s * ROWS_PER_TILE
    for j in range(ROWS_PER_TILE // CHUNK):
        pltpu.sync_copy(xg_v.at[0], aggr_sh.at[pl.ds(nbase + j * CHUNK, CHUNK)])
    rem = ROWS_PER_TILE % CHUNK
    if rem:
        pltpu.sync_copy(
            xg_v.at[0].at[pl.ds(0, rem)],
            aggr_sh.at[pl.ds(nbase + (ROWS_PER_TILE // CHUNK) * CHUNK, rem)])
    plsc.subcore_barrier()

    ebase = wid * E_PER_W

    ksel = [jnp.full((F,), k, jnp.int32) for k in range(2 * FA)]

    def compute_chunk(b):
        def pair_body(p2, _):
            for u in range(2):
                p = 2 * p2 + u
                a2 = attr_b[b][pl.ds(p * (2 * FA), 2 * FA)]
                i0 = 2 * p
                i1 = 2 * p + 1
                acc0 = xg_v[b, i0] + bvec
                acc1 = xg_v[b, i1] + bvec
                for k in range(FA):
                    acc0 = acc0 + a2[ksel[k]] * wr[k]
                    acc1 = acc1 + a2[ksel[FA + k]] * wr[k]
                xg_v[b, i0] = jnp.maximum(acc0, 0.0)
                xg_v[b, i1] = jnp.maximum(acc1, 0.0)
            return 0

        lax.fori_loop(0, PAIRS // 2, pair_body, 0)

    for d in _in_copies(eidx_hbm, attr_hbm, src_b[0], dst_b[0],
                        attr_b[0], sem_in[0], ebase):
        d.start()
    for d in _in_copies(eidx_hbm, attr_hbm, src_b[0], dst_b[0],
                        attr_b[0], sem_in[0], ebase):
        d.wait()
    pltpu.make_async_copy(x_hbm.at[src_b[0]], xg_v.at[0], sem_g[0]).start()
    for d in _in_copies(eidx_hbm, attr_hbm, src_b[1], dst_b[1],
                        attr_b[1], sem_in[1], ebase + CHUNK):
        d.start()

    def outer_body(g2, _):
        for b in range(2):
            g = 2 * g2 + b
            off = ebase + g * CHUNK
            bo = 1 - b

            @pl.when(g + 1 < N_CHUNKS)
            def _():
                for d in _in_copies(eidx_hbm, attr_hbm, src_b[bo], dst_b[bo],
                                    attr_b[bo], sem_in[bo], off + CHUNK):
                    d.wait()
                pltpu.make_async_copy(
                    x_hbm.at[src_b[bo]], xg_v.at[bo], sem_g[bo]).start()
            pltpu.make_async_copy(
                x_hbm.at[src_b[b]], xg_v.at[b], sem_g[b]).wait()
            compute_chunk(b)
            pltpu.sync_copy(xg_v.at[b], aggr_sh.at[dst_b[b]], add=True)
            @pl.when(g + 2 < N_CHUNKS)
            def _():
                for d in _in_copies(eidx_hbm, attr_hbm, src_b[b], dst_b[b],
                                    attr_b[b], sem_in[b], off + 2 * CHUNK):
                    d.start()
        return 0

    lax.fori_loop(0, N_CHUNKS // 2, outer_body, 0)
    plsc.subcore_barrier()

    pltpu.sync_copy(aggr_sh.at[pl.ds(nbase, ROWS_PER_TILE)],
                    out_hbm.at[c].at[pl.ds(nbase, ROWS_PER_TILE)])


_sc_aggregate = functools.partial(
    pl.kernel,
    out_type=jax.ShapeDtypeStruct((NC, N_PAD, F), jnp.float32),
    mesh=plsc.VectorSubcoreMesh(core_axis_name="c", subcore_axis_name="s",
                                num_cores=NC, num_subcores=NS),
    compiler_params=pltpu.CompilerParams(use_tc_tiling_on_sc=False),
    scratch_types=[
        pltpu.VMEM((CHUNK,), jnp.int32),
        pltpu.VMEM((CHUNK,), jnp.int32),
        pltpu.VMEM((CHUNK,), jnp.int32),
        pltpu.VMEM((CHUNK,), jnp.int32),
        pltpu.VMEM((CHUNK * FA,), jnp.float32),
        pltpu.VMEM((CHUNK * FA,), jnp.float32),
        pltpu.VMEM((2, CHUNK, F), jnp.float32),
        pltpu.VMEM((FA + 1, F), jnp.float32),
        pltpu.VMEM_SHARED((N_PAD, F), jnp.float32),
        pltpu.SemaphoreType.DMA,
        pltpu.SemaphoreType.DMA,
        pltpu.SemaphoreType.DMA,
        pltpu.SemaphoreType.DMA,
    ],
)(_sc_body)


def kernel(x, edge_index, edge_attr, lin1_W, lin1_b, nn_W, nn_b):
    eidx = edge_index.astype(jnp.int32)
    attr_flat = edge_attr.reshape(E * FA)
    aggr2 = _sc_aggregate(eidx, attr_flat, x, lin1_W.T, lin1_b)[:, :N, :]
    out = _final(x, aggr2, nn_W.T, nn_b.reshape(1, FO))
    return out


# speedup vs baseline: 13.6588x; 1.5167x over previous
import functools

import jax
import jax.numpy as jnp
from jax import lax
from jax.experimental import pallas as pl
from jax.experimental.pallas import tpu as pltpu
from jax.experimental.pallas import tpu_sc as plsc

N = 100000
E = 3200000
F = 16
FA = 8
FO = 32
BL = 128
NBLK = E // BL

NC = 2
NS = 16
NW = NC * NS

NB_LO = NBLK // NW
NB_EXTRA = NBLK % NW
CB = 4
N_MAIN = 194

ROWS_PER_TILE = 6256
N_PAD = ROWS_PER_TILE * NS

BN = 10000


def _final_body(x_ref, a_ref, w_ref, b_ref, o_ref):
    h = x_ref[...] + a_ref[0] + a_ref[1]
    o_ref[...] = (
        lax.dot_general(h, w_ref[...], (((1,), (0,)), ((), ())),
                        preferred_element_type=jnp.float32)
        + b_ref[...]
    )


def _final(x, aggr2, w_t, b):
    return pl.pallas_call(
        _final_body,
        grid=(N // BN,),
        in_specs=[
            pl.BlockSpec((BN, F), lambda i: (i, 0)),
            pl.BlockSpec((NC, BN, F), lambda i: (0, i, 0)),
            pl.BlockSpec((F, FO), lambda i: (0, 0)),
            pl.BlockSpec((1, FO), lambda i: (0, 0)),
        ],
        out_specs=pl.BlockSpec((BN, FO), lambda i: (i, 0)),
        out_shape=jax.ShapeDtypeStruct((N, FO), jnp.float32),
    )(x, aggr2, w_t, b)


def _sc_body(eidx_hbm, attr_hbm, x_hbm, w_hbm, b_hbm, out_hbm,
             ei0_v, ei1_v, at0_v, at1_v, xg_v, wb_v, aggr_sh,
             sem_in0, sem_in1, sem_g0, sem_g1):
    c = lax.axis_index("c")
    s = lax.axis_index("s")
    wid = c * NS + s
    sem_in = (sem_in0, sem_in1)
    sem_g = (sem_g0, sem_g1)
    ei_b = (ei0_v, ei1_v)
    at_b = (at0_v, at1_v)

    pltpu.sync_copy(w_hbm, wb_v.at[pl.ds(0, FA)])
    pltpu.sync_copy(b_hbm, wb_v.at[FA])
    wr = [wb_v[k] for k in range(FA)]
    bvec = wb_v[FA]

    lsel = [jnp.full((F,), l, jnp.int32) for l in range(F)]

    def zero_body(i, _):
        xg_v[0, i] = jnp.zeros((F,), jnp.float32)
        return 0

    lax.fori_loop(0, CB * BL, zero_body, 0)
    nbase = s * ROWS_PER_TILE
    zrows = CB * BL
    for j in range(ROWS_PER_TILE // zrows):
        pltpu.sync_copy(xg_v.at[0],
                        aggr_sh.at[pl.ds(nbase + j * zrows, zrows)])
    rem = ROWS_PER_TILE % zrows
    if rem:
        pltpu.sync_copy(
            xg_v.at[0].at[pl.ds(0, rem)],
            aggr_sh.at[pl.ds(nbase + (ROWS_PER_TILE // zrows) * zrows, rem)])
    plsc.subcore_barrier()

    bb = NB_LO * wid + jnp.minimum(wid, NB_EXTRA)

    def in_copies(b, bs, nb):
        return (
            pltpu.make_async_copy(
                eidx_hbm.at[pl.ds(bs, nb)], ei_b[b].at[pl.ds(0, nb)],
                sem_in[b]),
            pltpu.make_async_copy(
                attr_hbm.at[pl.ds(bs, nb)], at_b[b].at[pl.ds(0, nb)],
                sem_in[b]),
        )

    def start_gathers(b, nb):
        for bl in range(nb):
            pltpu.make_async_copy(
                x_hbm.at[ei_b[b].at[bl, 0]],
                xg_v.at[b].at[pl.ds(bl * BL, BL)], sem_g[b]).start()

    def wait_gathers(b, nb):
        for bl in range(nb):
            pltpu.make_async_copy(
                x_hbm.at[ei_b[b].at[bl, 0]],
                xg_v.at[b].at[pl.ds(bl * BL, BL)], sem_g[b]).wait()

    def compute_blocks(b, nb):
        def group_body(qq, _):
            bl = qq // 8
            q = qq % 8
            ak = [at_b[b][bl, k, pl.ds(q * F, F)] for k in range(FA)]
            ebase = qq * F
            for l in range(F):
                i = ebase + l
                acc = xg_v[b, i] + bvec
                for k in range(FA):
                    acc = acc + ak[k][lsel[l]] * wr[k]
                xg_v[b, i] = jnp.maximum(acc, 0.0)
            return 0

        lax.fori_loop(0, nb * 8, group_body, 0)

    def scatter_blocks(b, nb):
        for bl in range(nb):
            pltpu.sync_copy(xg_v.at[b].at[pl.ds(bl * BL, BL)],
                            aggr_sh.at[ei_b[b].at[bl, 1]], add=True)

    for d in in_copies(0, bb, CB):
        d.start()
    for d in in_copies(0, bb, CB):
        d.wait()
    start_gathers(0, CB)
    for d in in_copies(1, bb + CB, CB):
        d.start()

    def outer_body(g2, _):
        for b in range(2):
            g = 2 * g2 + b
            bs = bb + g * CB
            bo = 1 - b

            @pl.when(g + 1 < N_MAIN)
            def _():
                for d in in_copies(bo, bs + CB, CB):
                    d.wait()
                start_gathers(bo, CB)
            wait_gathers(b, CB)
            compute_blocks(b, CB)
            scatter_blocks(b, CB)

            @pl.when(g + 2 < N_MAIN)
            def _():
                for d in in_copies(b, bs + 2 * CB, CB):
                    d.start()
        return 0

    lax.fori_loop(0, N_MAIN // 2, outer_body, 0)

    def sync_chunk(bs, nb):
        for d in in_copies(0, bs, nb):
            d.start()
        for d in in_copies(0, bs, nb):
            d.wait()
        start_gathers(0, nb)
        wait_gathers(0, nb)
        compute_blocks(0, nb)
        scatter_blocks(0, nb)

    sync_chunk(bb + N_MAIN * CB, CB)

    @pl.when(wid < NB_EXTRA)
    def _():
        sync_chunk(bb + N_MAIN * CB + CB, 2)

    @pl.when(wid >= NB_EXTRA)
    def _():
        sync_chunk(bb + N_MAIN * CB + CB, 1)

    plsc.subcore_barrier()

    pltpu.sync_copy(aggr_sh.at[pl.ds(nbase, ROWS_PER_TILE)],
                    out_hbm.at[c].at[pl.ds(nbase, ROWS_PER_TILE)])


_sc_aggregate = functools.partial(
    pl.kernel,
    out_type=jax.ShapeDtypeStruct((NC, N_PAD, F), jnp.float32),
    mesh=plsc.VectorSubcoreMesh(core_axis_name="c", subcore_axis_name="s",
                                num_cores=NC, num_subcores=NS),
    compiler_params=pltpu.CompilerParams(use_tc_tiling_on_sc=False),
    scratch_types=[
        pltpu.VMEM((CB, 2, BL), jnp.int32),
        pltpu.VMEM((CB, 2, BL), jnp.int32),
        pltpu.VMEM((CB, FA, BL), jnp.float32),
        pltpu.VMEM((CB, FA, BL), jnp.float32),
        pltpu.VMEM((2, CB * BL, F), jnp.float32),
        pltpu.VMEM((FA + 1, F), jnp.float32),
        pltpu.VMEM_SHARED((N_PAD, F), jnp.float32),
        pltpu.SemaphoreType.DMA,
        pltpu.SemaphoreType.DMA,
        pltpu.SemaphoreType.DMA,
        pltpu.SemaphoreType.DMA,
    ],
)(_sc_body)


def kernel(x, edge_index, edge_attr, lin1_W, lin1_b, nn_W, nn_b):
    eidx3 = (edge_index.astype(jnp.int32)
             .reshape(2, NBLK, BL).transpose(1, 0, 2))
    attr3 = edge_attr.reshape(NBLK, BL, FA).transpose(0, 2, 1)
    aggr2 = _sc_aggregate(eidx3, attr3, x, lin1_W.T, lin1_b)
    out = _final(x, aggr2[:, :N, :], nn_W.T, nn_b.reshape(1, FO))
    return out


# speedup vs baseline: 14.1853x; 1.0385x over previous
import functools

import jax
import jax.numpy as jnp
from jax import lax
from jax.experimental import pallas as pl
from jax.experimental.pallas import tpu as pltpu
from jax.experimental.pallas import tpu_sc as plsc

N = 100000
E = 3200000
F = 16
FA = 8
FO = 32
BL = 128
NBLK = E // BL

NC = 2
NS = 16
NW = NC * NS

NB_LO = NBLK // NW
NB_EXTRA = NBLK % NW
CB = 4
N_MAIN = 194

ROWS_PER_TILE = 6256
N_PAD = ROWS_PER_TILE * NS

BN = 10000


def _final_body(x_ref, a_ref, w_ref, b_ref, o_ref):
    h = x_ref[...] + a_ref[0] + a_ref[1]
    o_ref[...] = (
        lax.dot_general(h, w_ref[...], (((1,), (0,)), ((), ())),
                        preferred_element_type=jnp.float32)
        + b_ref[...]
    )


def _final(x, aggr2, w_t, b):
    return pl.pallas_call(
        _final_body,
        grid=(N // BN,),
        in_specs=[
            pl.BlockSpec((BN, F), lambda i: (i, 0)),
            pl.BlockSpec((NC, BN, F), lambda i: (0, i, 0)),
            pl.BlockSpec((F, FO), lambda i: (0, 0)),
            pl.BlockSpec((1, FO), lambda i: (0, 0)),
        ],
        out_specs=pl.BlockSpec((BN, FO), lambda i: (i, 0)),
        out_shape=jax.ShapeDtypeStruct((N, FO), jnp.float32),
    )(x, aggr2, w_t, b)


def _sc_body(eidx_hbm, attr_hbm, x_hbm, w_hbm, b_hbm, out_hbm,
             ei0_v, ei1_v, at0_v, at1_v, xg_v, wb_v, aggr_sh,
             sem_in0, sem_in1, sem_g0, sem_g1):
    c = lax.axis_index("c")
    s = lax.axis_index("s")
    wid = c * NS + s
    sem_in = (sem_in0, sem_in1)
    sem_g = (sem_g0, sem_g1)
    ei_b = (ei0_v, ei1_v)
    at_b = (at0_v, at1_v)

    pltpu.sync_copy(w_hbm, wb_v.at[pl.ds(0, FA)])
    pltpu.sync_copy(b_hbm, wb_v.at[FA])
    wr = [wb_v[k] for k in range(FA)]
    bvec = wb_v[FA]

    lsel = [jnp.full((F,), l, jnp.int32) for l in range(F)]

    def zero_body(i, _):
        xg_v[0, i] = jnp.zeros((F,), jnp.float32)
        return 0

    lax.fori_loop(0, CB * BL, zero_body, 0)
    nbase = s * ROWS_PER_TILE
    zrows = CB * BL
    for j in range(ROWS_PER_TILE // zrows):
        pltpu.sync_copy(xg_v.at[0],
                        aggr_sh.at[pl.ds(nbase + j * zrows, zrows)])
    rem = ROWS_PER_TILE % zrows
    if rem:
        pltpu.sync_copy(
            xg_v.at[0].at[pl.ds(0, rem)],
            aggr_sh.at[pl.ds(nbase + (ROWS_PER_TILE // zrows) * zrows, rem)])
    plsc.subcore_barrier()

    bb = NB_LO * wid + jnp.minimum(wid, NB_EXTRA)

    def in_copies(b, bs, nb):
        return (
            pltpu.make_async_copy(
                eidx_hbm.at[pl.ds(bs, nb)], ei_b[b].at[pl.ds(0, nb)],
                sem_in[b]),
            pltpu.make_async_copy(
                attr_hbm.at[pl.ds(bs, nb)], at_b[b].at[pl.ds(0, nb)],
                sem_in[b]),
        )

    def start_gathers(b, nb):
        for bl in range(nb):
            pltpu.make_async_copy(
                x_hbm.at[ei_b[b].at[bl, 0]],
                xg_v.at[b].at[pl.ds(bl * BL, BL)], sem_g[b]).start()

    def wait_gathers(b, nb):
        for bl in range(nb):
            pltpu.make_async_copy(
                x_hbm.at[ei_b[b].at[bl, 0]],
                xg_v.at[b].at[pl.ds(bl * BL, BL)], sem_g[b]).wait()

    def compute_blocks(b, nb):
        def group_body(qq, _):
            bl = qq // 8
            q = qq % 8
            ak = [at_b[b][bl, k, pl.ds(q * F, F)] for k in range(FA)]
            ebase = qq * F
            for l in range(F):
                i = ebase + l
                acc = xg_v[b, i] + bvec
                for k in range(FA):
                    acc = acc + ak[k][lsel[l]] * wr[k]
                xg_v[b, i] = jnp.maximum(acc, 0.0)
            return 0

        lax.fori_loop(0, nb * 8, group_body, 0)

    def scatter_blocks(b, nb):
        for bl in range(nb):
            pltpu.sync_copy(xg_v.at[b].at[pl.ds(bl * BL, BL)],
                            aggr_sh.at[ei_b[b].at[bl, 1]], add=True)

    for d in in_copies(0, bb, CB):
        d.start()
    for d in in_copies(0, bb, CB):
        d.wait()
    start_gathers(0, CB)
    for d in in_copies(1, bb + CB, CB):
        d.start()

    def outer_body(g2, _):
        for b in range(2):
            g = 2 * g2 + b
            bs = bb + g * CB
            bo = 1 - b

            @pl.when(g + 1 < N_MAIN)
            def _():
                for d in in_copies(bo, bs + CB, CB):
                    d.wait()
                start_gathers(bo, CB)
            wait_gathers(b, CB)
            compute_blocks(b, CB)
            scatter_blocks(b, CB)

            @pl.when(g + 2 < N_MAIN)
            def _():
                for d in in_copies(b, bs + 2 * CB, CB):
                    d.start()
        return 0

    lax.fori_loop(0, N_MAIN // 2, outer_body, 0)

    def sync_chunk(bs, nb):
        for d in in_copies(0, bs, nb):
            d.start()
        for d in in_copies(0, bs, nb):
            d.wait()
        start_gathers(0, nb)
        wait_gathers(0, nb)
        compute_blocks(0, nb)
        scatter_blocks(0, nb)

    sync_chunk(bb + N_MAIN * CB, CB)

    @pl.when(wid < NB_EXTRA)
    def _():
        sync_chunk(bb + N_MAIN * CB + CB, 2)

    @pl.when(wid >= NB_EXTRA)
    def _():
        sync_chunk(bb + N_MAIN * CB + CB, 1)

    plsc.subcore_barrier()

    pltpu.sync_copy(aggr_sh.at[pl.ds(nbase, ROWS_PER_TILE)],
                    out_hbm.at[c].at[pl.ds(nbase, ROWS_PER_TILE)])


_sc_aggregate = functools.partial(
    pl.kernel,
    out_type=jax.ShapeDtypeStruct((NC, N_PAD, F), jnp.float32),
    mesh=plsc.VectorSubcoreMesh(core_axis_name="c", subcore_axis_name="s",
                                num_cores=NC, num_subcores=NS),
    compiler_params=pltpu.CompilerParams(use_tc_tiling_on_sc=False),
    scratch_types=[
        pltpu.VMEM((CB, 2, BL), jnp.int32),
        pltpu.VMEM((CB, 2, BL), jnp.int32),
        pltpu.VMEM((CB, FA, BL), jnp.float32),
        pltpu.VMEM((CB, FA, BL), jnp.float32),
        pltpu.VMEM((2, CB * BL, F), jnp.float32),
        pltpu.VMEM((FA + 1, F), jnp.float32),
        pltpu.VMEM_SHARED((N_PAD, F), jnp.float32),
        pltpu.SemaphoreType.DMA,
        pltpu.SemaphoreType.DMA,
        pltpu.SemaphoreType.DMA,
        pltpu.SemaphoreType.DMA,
    ],
)(_sc_body)


def kernel(x, edge_index, edge_attr, lin1_W, lin1_b, nn_W, nn_b):
    eidx3 = (edge_index.astype(jnp.int32)
             .reshape(2, NBLK, BL).transpose(1, 0, 2))
    attr3 = edge_attr.reshape(NBLK, BL, FA).transpose(0, 2, 1)
    aggr2 = _sc_aggregate(eidx3, attr3, x, lin1_W.T, lin1_b)
    out = _final(x, aggr2, nn_W.T, nn_b.reshape(1, FO))
    return out
